# baseline (speedup 1.0000x reference)
import functools

import jax
import jax.numpy as jnp
from jax import lax
from jax.experimental import pallas as pl
from jax.experimental.pallas import tpu as pltpu
from jax.experimental.pallas import tpu_sc as plsc

NC = 2
NS = 16
LANES = 16
STEP = 80
NB = 3


def _xa_matmul(x, w):
    n = x.shape[0]
    blk = 2000
    grid = n // blk

    def body(x_ref, w_ref, o_ref):
        o_ref[...] = jnp.dot(x_ref[...], w_ref[...],
                             preferred_element_type=jnp.float32)

    return pl.pallas_call(
        body,
        grid=(grid,),
        in_specs=[
            pl.BlockSpec((blk, x.shape[1]), lambda i: (i, 0)),
            pl.BlockSpec(w.shape, lambda i: (0, 0)),
        ],
        out_specs=pl.BlockSpec((blk, w.shape[1]), lambda i: (i, 0)),
        out_shape=jax.ShapeDtypeStruct((n, w.shape[1]), jnp.float32),
    )(x, w)


def _ea_matmul(edge_attr, w, b):
    e, da = edge_attr.shape
    dh = w.shape[1]
    pk = 128 // da
    a_pack = edge_attr.reshape(e // pk, pk * da)
    w_bd = jnp.kron(jnp.eye(pk, dtype=w.dtype), w)
    b_t = jnp.tile(b, (1, pk))
    blk = 2000
    grid = (e // pk) // blk

    def body(a_ref, w_ref, b_ref, o_ref):
        o_ref[...] = jnp.dot(a_ref[...], w_ref[...],
                             preferred_element_type=jnp.float32) + b_ref[...]

    return pl.pallas_call(
        body,
        grid=(grid,),
        in_specs=[
            pl.BlockSpec((blk, pk * da), lambda i: (i, 0)),
            pl.BlockSpec(w_bd.shape, lambda i: (0, 0)),
            pl.BlockSpec(b_t.shape, lambda i: (0, 0)),
        ],
        out_specs=pl.BlockSpec((blk, pk * dh), lambda i: (i, 0)),
        out_shape=jax.ShapeDtypeStruct((e // pk, pk * dh), jnp.float32),
    )(a_pack, w_bd, b_t)


def _sc_segment_sum(xa, ea_pack, row, col):
    n, d = xa.shape
    epk = ea_pack.shape[1] // d
    e = ea_pack.shape[0] * epk
    nw = NC * NS
    nchunks = e // STEP
    nfull = nchunks // nw
    rem = nchunks - nfull * nw
    zc = 64
    npad = -(-n // (NS * zc)) * (NS * zc)
    rpt = npad // NS
    nz = rpt // zc

    mesh = plsc.VectorSubcoreMesh(core_axis_name="c", subcore_axis_name="s",
                                  num_cores=NC, num_subcores=NS)

    dw = d + LANES

    @functools.partial(
        pl.kernel,
        out_type=[
            jax.ShapeDtypeStruct((NC, npad, dw), jnp.float32),
        ],
        mesh=mesh,
        scratch_types=[
            pltpu.VMEM_SHARED((npad, dw), jnp.float32),
            pltpu.VMEM((NB, STEP), jnp.int32),
            pltpu.VMEM((NB, STEP), jnp.int32),
            pltpu.VMEM((NB, STEP), jnp.int32),
            pltpu.VMEM((NB, STEP // epk, epk * d), jnp.float32),
            pltpu.VMEM((NB, STEP, d), jnp.float32),
            pltpu.VMEM((NB, STEP, dw), jnp.float32),
            pltpu.VMEM((zc, dw), jnp.float32),
        ] + [pltpu.SemaphoreType.DMA] * (3 * NB + 1),
        compiler_params=pltpu.CompilerParams(use_tc_tiling_on_sc=False),
    )
    def sc_kernel(xa_hbm, ea_hbm, row_hbm, col_hbm, acc_out,
                  acc_sh, rbuf, cbuf, cbuf2, eabuf, gbuf, sbuf, zbuf,
                  *sems):
        seml = sems[0:NB]
        semg = sems[NB:2 * NB]
        semsc = sems[2 * NB:3 * NB]
        sem = sems[3 * NB]
        cid = lax.axis_index("c")
        sid = lax.axis_index("s")
        dv = d // LANES

        zeros = jnp.zeros((LANES,), jnp.float32)
        ones = jnp.ones((LANES,), jnp.float32)

        def fill_z(i, _):
            for j in range(dw // LANES):
                zbuf[i, pl.ds(j * LANES, LANES)] = zeros
            return 0

        lax.fori_loop(0, zc, fill_z, 0)

        def fill_o(i, _):
            for b in range(NB):
                sbuf[b, i, pl.ds(d, LANES)] = ones
            return 0

        lax.fori_loop(0, STEP, fill_o, 0)

        for k in range(nz):
            roff = sid * rpt + k * zc
            pltpu.sync_copy(zbuf, acc_sh.at[pl.ds(roff, zc)])
        plsc.subcore_barrier()

        w = sid * NC + cid
        nfp = (nfull // NB) * NB

        sea = STEP // epk

        def start_load(k, b):
            c = w + nw * k
            off = pl.multiple_of(c * STEP, STEP)
            eoff = pl.multiple_of(c * sea, sea)
            pltpu.async_copy(row_hbm.at[pl.ds(off, STEP)], rbuf.at[b], seml[b])
            pltpu.async_copy(col_hbm.at[pl.ds(off, STEP)], cbuf.at[b], seml[b])
            pltpu.async_copy(ea_hbm.at[pl.ds(eoff, sea)], eabuf.at[b],
                             seml[b])

        def wait_load(b):
            pltpu.make_async_copy(row_hbm.at[pl.ds(0, STEP)], rbuf.at[b],
                                  seml[b]).wait()
            pltpu.make_async_copy(col_hbm.at[pl.ds(0, STEP)], cbuf.at[b],
                                  seml[b]).wait()
            pltpu.make_async_copy(ea_hbm.at[pl.ds(0, sea)], eabuf.at[b],
                                  seml[b]).wait()

        def start_gather(b):
            pltpu.async_copy(xa_hbm.at[rbuf.at[b]], gbuf.at[b], semg[b])

        def wait_gather(b):
            pltpu.make_async_copy(xa_hbm.at[rbuf.at[b]], gbuf.at[b],
                                  semg[b]).wait()

        def compute(b):
            def crow(q):
                for p in range(epk):
                    i = q * epk + p
                    for jj in range(dv):
                        sl = pl.ds(jj * LANES, LANES)
                        sbuf[b, i, sl] = jnp.maximum(
                            gbuf[b, i, sl]
                            + eabuf[b, q, pl.ds(p * d + jj * LANES, LANES)],
                            0.0)

            plsc.parallel_loop(0, sea, unroll=2)(crow)

            def ccopy(i):
                sl = pl.ds(i, LANES)
                cbuf2[b, sl] = cbuf[b, sl]

            plsc.parallel_loop(0, STEP, step=LANES)(ccopy)

        def start_scatter(b):
            pltpu.async_copy(sbuf.at[b], acc_sh.at[cbuf2.at[b]], semsc[b],
                             add=True)

        def wait_scatter(b):
            pltpu.make_async_copy(sbuf.at[b], acc_sh.at[cbuf2.at[b]],
                                  semsc[b]).wait()

        def group(g, first):
            for b in range(NB):
                if not first:
                    wait_scatter(b)
                wait_load(b)
                start_gather(b)
            for b in range(NB):
                wait_gather(b)
                compute(b)
                start_scatter(b)

                @pl.when(g + NB + b < nfp)
                def _():
                    start_load(g + NB + b, b)

        if nfp >= NB:
            for b in range(NB):
                start_load(b, b)
            group(0, True)
            if nfp > NB:
                pl.loop(NB, nfp, step=NB)(lambda g: group(g, False))
            for b in range(NB):
                wait_scatter(b)

        for k in range(nfp, nfull):
            start_load(k, 0)
            wait_load(0)
            start_gather(0)
            wait_gather(0)
            compute(0)
            start_scatter(0)
            wait_scatter(0)

        if rem:
            @pl.when(w < rem)
            def _():
                start_load(nfull, 0)
                wait_load(0)
                start_gather(0)
                wait_gather(0)
                compute(0)
                start_scatter(0)
                wait_scatter(0)

        plsc.subcore_barrier()
        for k in range(nz):
            roff = sid * rpt + k * zc
            pltpu.sync_copy(acc_sh.at[pl.ds(roff, zc)],
                            acc_out.at[cid, pl.ds(roff, zc)])

    (acc_p,) = sc_kernel(xa, ea_pack, row, col)
    return acc_p


def _node_mlp(x, acc_p, batch2d, u, W1b, b1b2, W2a, b2a2, W2b, b2b2):
    n, dn = x.shape
    d = W1b.shape[0]
    dw = acc_p.shape[2]
    nb = u.shape[0]
    dout = W2b.shape[1]
    blk = 2000
    grid = n // blk

    def body(x_ref, acc_ref, b_ref, u_ref, w1b_ref, b1b_ref,
             w2a_ref, b2a_ref, w2b_ref, b2b_ref, o_ref):
        sr = acc_ref[0, :, 0:d] + acc_ref[1, :, 0:d]
        cnt = acc_ref[0, :, d:d + 1] + acc_ref[1, :, d:d + 1]
        mean = sr / jnp.maximum(cnt, 1.0)
        mask = jnp.minimum(cnt, 1.0)
        w2a_x = w2a_ref[0:dn, :]
        w2a_m = w2a_ref[dn:dn + d, :]
        w2a_u = w2a_ref[dn + d:, :]
        wm = jnp.dot(w1b_ref[...], w2a_m, preferred_element_type=jnp.float32)
        bm = jnp.dot(b1b_ref[...], w2a_m, preferred_element_type=jnp.float32)
        up = jnp.dot(u_ref[...], w2a_u, preferred_element_type=jnp.float32)
        oh = (b_ref[...] == lax.broadcasted_iota(jnp.int32, (blk, nb), 1)
              ).astype(jnp.float32)
        pre = (jnp.dot(x_ref[...], w2a_x, preferred_element_type=jnp.float32)
               + jnp.dot(mean, wm, preferred_element_type=jnp.float32)
               + mask * bm
               + jnp.dot(oh, up, preferred_element_type=jnp.float32)
               + b2a_ref[...])
        h2 = jnp.maximum(pre, 0.0)
        o_ref[...] = jnp.dot(h2, w2b_ref[...],
                             preferred_element_type=jnp.float32) + b2b_ref[...]

    return pl.pallas_call(
        body,
        grid=(grid,),
        in_specs=[
            pl.BlockSpec((blk, dn), lambda i: (i, 0)),
            pl.BlockSpec((2, blk, dw), lambda i: (0, i, 0)),
            pl.BlockSpec((blk, 1), lambda i: (i, 0)),
            pl.BlockSpec(u.shape, lambda i: (0, 0)),
            pl.BlockSpec(W1b.shape, lambda i: (0, 0)),
            pl.BlockSpec(b1b2.shape, lambda i: (0, 0)),
            pl.BlockSpec(W2a.shape, lambda i: (0, 0)),
            pl.BlockSpec(b2a2.shape, lambda i: (0, 0)),
            pl.BlockSpec(W2b.shape, lambda i: (0, 0)),
            pl.BlockSpec(b2b2.shape, lambda i: (0, 0)),
        ],
        out_specs=pl.BlockSpec((blk, dout), lambda i: (i, 0)),
        out_shape=jax.ShapeDtypeStruct((n, dout), jnp.float32),
    )(x, acc_p, batch2d, u, W1b, b1b2, W2a, b2a2, W2b, b2b2)


def kernel(x, edge_index, edge_attr, u, batch,
           W1a, b1a, W1b, b1b, W2a, b2a, W2b, b2b):
    dn = x.shape[1]
    row = edge_index[0].astype(jnp.int32)
    col = edge_index[1].astype(jnp.int32)

    xa = _xa_matmul(x, W1a[:dn])
    ea = _ea_matmul(edge_attr, W1a[dn:], b1a.reshape(1, -1))
    acc_p = _sc_segment_sum(xa, ea, row, col)
    out = _node_mlp(x, acc_p, batch.astype(jnp.int32).reshape(-1, 1),
                    u, W1b, b1b.reshape(1, -1), W2a, b2a.reshape(1, -1),
                    W2b, b2b.reshape(1, -1))
    return out


# speedup vs baseline: 1.1693x; 1.1693x over previous
import functools

import jax
import jax.numpy as jnp
from jax import lax
from jax.experimental import pallas as pl
from jax.experimental.pallas import tpu as pltpu
from jax.experimental.pallas import tpu_sc as plsc

NC = 2
NS = 16
LANES = 16
STEP = 128
NB = 2


def _xa_matmul(x, w):
    n = x.shape[0]
    blk = 2000
    grid = n // blk

    def body(x_ref, w_ref, o_ref):
        o_ref[...] = jnp.dot(x_ref[...], w_ref[...],
                             preferred_element_type=jnp.float32)

    return pl.pallas_call(
        body,
        grid=(grid,),
        in_specs=[
            pl.BlockSpec((blk, x.shape[1]), lambda i: (i, 0)),
            pl.BlockSpec(w.shape, lambda i: (0, 0)),
        ],
        out_specs=pl.BlockSpec((blk, w.shape[1]), lambda i: (i, 0)),
        out_shape=jax.ShapeDtypeStruct((n, w.shape[1]), jnp.float32),
    )(x, w)


def _ea_matmul(edge_attr, w, b):
    e, da = edge_attr.shape
    dh = w.shape[1]
    pk = 128 // da
    a_pack = edge_attr.reshape(e // pk, pk * da)
    w_bd = jnp.kron(jnp.eye(pk, dtype=w.dtype), w)
    b_t = jnp.tile(b, (1, pk))
    blk = 2000
    grid = (e // pk) // blk

    def body(a_ref, w_ref, b_ref, o_ref):
        o_ref[...] = jnp.dot(a_ref[...], w_ref[...],
                             preferred_element_type=jnp.float32) + b_ref[...]

    out = pl.pallas_call(
        body,
        grid=(grid,),
        in_specs=[
            pl.BlockSpec((blk, pk * da), lambda i: (i, 0)),
            pl.BlockSpec(w_bd.shape, lambda i: (0, 0)),
            pl.BlockSpec(b_t.shape, lambda i: (0, 0)),
        ],
        out_specs=pl.BlockSpec((blk, pk * dh), lambda i: (i, 0)),
        out_shape=jax.ShapeDtypeStruct((e // pk, pk * dh), jnp.float32),
    )(a_pack, w_bd, b_t)
    return out.reshape(e, dh)


def _sc_segment_sum(xa, ea, row, col):
    n, d = xa.shape
    e = ea.shape[0]
    nw = NC * NS
    epw = e // nw
    nfull = epw // STEP
    tail = epw - nfull * STEP
    zc = 64
    npad = -(-n // (NS * zc)) * (NS * zc)
    rpt = npad // NS
    nz = rpt // zc

    mesh = plsc.VectorSubcoreMesh(core_axis_name="c", subcore_axis_name="s",
                                  num_cores=NC, num_subcores=NS)

    dw = d + LANES

    @functools.partial(
        pl.kernel,
        out_type=[
            jax.ShapeDtypeStruct((NC, npad, dw), jnp.float32),
        ],
        mesh=mesh,
        scratch_types=[
            pltpu.VMEM_SHARED((npad, dw), jnp.float32),
            pltpu.VMEM((NB, STEP), jnp.int32),
            pltpu.VMEM((NB, STEP), jnp.int32),
            pltpu.VMEM((NB, STEP), jnp.int32),
            pltpu.VMEM((NB, STEP, d), jnp.float32),
            pltpu.VMEM((NB, STEP, d), jnp.float32),
            pltpu.VMEM((NB, STEP, dw), jnp.float32),
            pltpu.VMEM((zc, dw), jnp.float32),
        ] + [pltpu.SemaphoreType.DMA] * (3 * NB + 1),
        compiler_params=pltpu.CompilerParams(use_tc_tiling_on_sc=False),
    )
    def sc_kernel(xa_hbm, ea_hbm, row_hbm, col_hbm, acc_out,
                  acc_sh, rbuf, cbuf, cbuf2, eabuf, gbuf, sbuf, zbuf,
                  *sems):
        seml = sems[0:NB]
        semg = sems[NB:2 * NB]
        semsc = sems[2 * NB:3 * NB]
        sem = sems[3 * NB]
        cid = lax.axis_index("c")
        sid = lax.axis_index("s")
        dv = d // LANES

        zeros = jnp.zeros((LANES,), jnp.float32)
        ones = jnp.ones((LANES,), jnp.float32)

        def fill_z(i, _):
            for j in range(dw // LANES):
                zbuf[i, pl.ds(j * LANES, LANES)] = zeros
            return 0

        lax.fori_loop(0, zc, fill_z, 0)

        def fill_o(i, _):
            for b in range(NB):
                sbuf[b, i, pl.ds(d, LANES)] = ones
            return 0

        lax.fori_loop(0, STEP, fill_o, 0)

        for k in range(nz):
            roff = sid * rpt + k * zc
            pltpu.sync_copy(zbuf, acc_sh.at[pl.ds(roff, zc)])
        plsc.subcore_barrier()

        w = sid * NC + cid
        base = w * epw
        nfp = (nfull // NB) * NB

        def start_load(j, b):
            off = base + j * STEP
            pltpu.async_copy(row_hbm.at[pl.ds(off, STEP)], rbuf.at[b], seml[b])
            pltpu.async_copy(col_hbm.at[pl.ds(off, STEP)], cbuf.at[b], seml[b])
            pltpu.async_copy(ea_hbm.at[pl.ds(off, STEP)], eabuf.at[b], seml[b])

        def wait_load(b):
            pltpu.make_async_copy(row_hbm.at[pl.ds(0, STEP)], rbuf.at[b],
                                  seml[b]).wait()
            pltpu.make_async_copy(col_hbm.at[pl.ds(0, STEP)], cbuf.at[b],
                                  seml[b]).wait()
            pltpu.make_async_copy(ea_hbm.at[pl.ds(0, STEP)], eabuf.at[b],
                                  seml[b]).wait()

        def start_gather(b):
            pltpu.async_copy(xa_hbm.at[rbuf.at[b]], gbuf.at[b], semg[b])

        def wait_gather(b):
            pltpu.make_async_copy(xa_hbm.at[rbuf.at[b]], gbuf.at[b],
                                  semg[b]).wait()

        def compute(b):
            def crow(i):
                for jj in range(dv):
                    sl = pl.ds(jj * LANES, LANES)
                    sbuf[b, i, sl] = jnp.maximum(
                        gbuf[b, i, sl] + eabuf[b, i, sl], 0.0)

            plsc.parallel_loop(0, STEP, unroll=4)(crow)

            def ccopy(i):
                sl = pl.ds(i, LANES)
                cbuf2[b, sl] = cbuf[b, sl]

            plsc.parallel_loop(0, STEP, step=LANES)(ccopy)

        def start_scatter(b):
            pltpu.async_copy(sbuf.at[b], acc_sh.at[cbuf2.at[b]], semsc[b],
                             add=True)

        def wait_scatter(b):
            pltpu.make_async_copy(sbuf.at[b], acc_sh.at[cbuf2.at[b]],
                                  semsc[b]).wait()

        def group(g, first):
            for b in range(NB):
                if not first:
                    wait_scatter(b)
                wait_load(b)
                start_gather(b)
            for b in range(NB):
                wait_gather(b)
                compute(b)
                start_scatter(b)

                @pl.when(g + NB + b < nfp)
                def _():
                    start_load(g + NB + b, b)

        if nfp >= NB:
            for b in range(NB):
                start_load(b, b)
            group(0, True)
            if nfp > NB:
                pl.loop(NB, nfp, step=NB)(lambda g: group(g, False))
            for b in range(NB):
                wait_scatter(b)

        for j in range(nfp, nfull):
            start_load(j, 0)
            wait_load(0)
            start_gather(0)
            wait_gather(0)
            compute(0)
            start_scatter(0)
            wait_scatter(0)

        if tail:
            off = base + nfull * STEP
            tsl = pl.ds(0, tail)
            pltpu.sync_copy(row_hbm.at[pl.ds(off, tail)], rbuf.at[0, tsl])
            pltpu.sync_copy(col_hbm.at[pl.ds(off, tail)], cbuf2.at[0, tsl])
            pltpu.sync_copy(ea_hbm.at[pl.ds(off, tail)], eabuf.at[0, tsl])
            pltpu.async_copy(xa_hbm.at[rbuf.at[0, tsl]], gbuf.at[0, tsl],
                             sem).wait()

            def trow(i, _):
                for jj in range(dv):
                    sl = pl.ds(jj * LANES, LANES)
                    sbuf[0, i, sl] = jnp.maximum(
                        gbuf[0, i, sl] + eabuf[0, i, sl], 0.0)
                return 0

            lax.fori_loop(0, tail, trow, 0)

            def tzero(i, _):
                for jj in range(dw // LANES):
                    sbuf[0, i, pl.ds(jj * LANES, LANES)] = zeros
                return 0

            lax.fori_loop(tail, STEP, tzero, 0)
            izeros = jnp.zeros((LANES,), jnp.int32)
            for i in range(tail, STEP, LANES):
                cbuf2[0, pl.ds(i, LANES)] = izeros
            pltpu.sync_copy(sbuf.at[0], acc_sh.at[cbuf2.at[0]], add=True)

        plsc.subcore_barrier()
        for k in range(nz):
            roff = sid * rpt + k * zc
            pltpu.sync_copy(acc_sh.at[pl.ds(roff, zc)],
                            acc_out.at[cid, pl.ds(roff, zc)])

    (acc_p,) = sc_kernel(xa, ea, row, col)
    return acc_p


def _node_mlp(x, acc_p, batch2d, u, W1b, b1b2, W2a, b2a2, W2b, b2b2):
    n, dn = x.shape
    d = W1b.shape[0]
    dw = acc_p.shape[2]
    nb = u.shape[0]
    dout = W2b.shape[1]
    blk = 2000
    grid = n // blk

    def body(x_ref, acc_ref, b_ref, u_ref, w1b_ref, b1b_ref,
             w2a_ref, b2a_ref, w2b_ref, b2b_ref, o_ref):
        sr = acc_ref[0, :, 0:d] + acc_ref[1, :, 0:d]
        cnt = acc_ref[0, :, d:d + 1] + acc_ref[1, :, d:d + 1]
        mean = sr / jnp.maximum(cnt, 1.0)
        mask = jnp.minimum(cnt, 1.0)
        w2a_x = w2a_ref[0:dn, :]
        w2a_m = w2a_ref[dn:dn + d, :]
        w2a_u = w2a_ref[dn + d:, :]
        wm = jnp.dot(w1b_ref[...], w2a_m, preferred_element_type=jnp.float32)
        bm = jnp.dot(b1b_ref[...], w2a_m, preferred_element_type=jnp.float32)
        up = jnp.dot(u_ref[...], w2a_u, preferred_element_type=jnp.float32)
        oh = (b_ref[...] == lax.broadcasted_iota(jnp.int32, (blk, nb), 1)
              ).astype(jnp.float32)
        pre = (jnp.dot(x_ref[...], w2a_x, preferred_element_type=jnp.float32)
               + jnp.dot(mean, wm, preferred_element_type=jnp.float32)
               + mask * bm
               + jnp.dot(oh, up, preferred_element_type=jnp.float32)
               + b2a_ref[...])
        h2 = jnp.maximum(pre, 0.0)
        o_ref[...] = jnp.dot(h2, w2b_ref[...],
                             preferred_element_type=jnp.float32) + b2b_ref[...]

    return pl.pallas_call(
        body,
        grid=(grid,),
        in_specs=[
            pl.BlockSpec((blk, dn), lambda i: (i, 0)),
            pl.BlockSpec((2, blk, dw), lambda i: (0, i, 0)),
            pl.BlockSpec((blk, 1), lambda i: (i, 0)),
            pl.BlockSpec(u.shape, lambda i: (0, 0)),
            pl.BlockSpec(W1b.shape, lambda i: (0, 0)),
            pl.BlockSpec(b1b2.shape, lambda i: (0, 0)),
            pl.BlockSpec(W2a.shape, lambda i: (0, 0)),
            pl.BlockSpec(b2a2.shape, lambda i: (0, 0)),
            pl.BlockSpec(W2b.shape, lambda i: (0, 0)),
            pl.BlockSpec(b2b2.shape, lambda i: (0, 0)),
        ],
        out_specs=pl.BlockSpec((blk, dout), lambda i: (i, 0)),
        out_shape=jax.ShapeDtypeStruct((n, dout), jnp.float32),
    )(x, acc_p, batch2d, u, W1b, b1b2, W2a, b2a2, W2b, b2b2)


def kernel(x, edge_index, edge_attr, u, batch,
           W1a, b1a, W1b, b1b, W2a, b2a, W2b, b2b):
    dn = x.shape[1]
    row = edge_index[0].astype(jnp.int32)
    col = edge_index[1].astype(jnp.int32)

    xa = _xa_matmul(x, W1a[:dn])
    ea = _ea_matmul(edge_attr, W1a[dn:], b1a.reshape(1, -1))
    acc_p = _sc_segment_sum(xa, ea, row, col)
    out = _node_mlp(x, acc_p, batch.astype(jnp.int32).reshape(-1, 1),
                    u, W1b, b1b.reshape(1, -1), W2a, b2a.reshape(1, -1),
                    W2b, b2b.reshape(1, -1))
    return out
